# Initial kernel scaffold; baseline (speedup 1.0000x reference)
import jax
import jax.numpy as jnp
from jax.experimental import pallas as pl


def kernel(x, aw1_0, ab1_0, aw2_0, ab2_0, aw1_1, ab1_1, aw2_1, ab2_1, cw0, cb0, cw1, cb1, bn_g0, bn_b0, bn_g1, bn_b1, clw, clb, edge_index):
    raise NotImplementedError("write your pallas kernel here")


# speedup vs baseline: 2.2259x; 2.2259x over previous
import functools

import jax
import jax.numpy as jnp
from jax import lax
from jax.experimental import pallas as pl
from jax.experimental.pallas import tpu as pltpu, tpu_sc as plsc

N = 10000
E = 320000
D = 128
H = 64

_SC_INFO = plsc.get_sparse_core_info()
NC = _SC_INFO.num_cores
NS = _SC_INFO.num_subcores
NW = NC * NS
CH = 80


def _proj_body(x_ref, wr_ref, wc_ref, bc_ref, p_ref, q_ref):
    x = x_ref[...]
    p_ref[...] = jnp.dot(x, wr_ref[...], preferred_element_type=jnp.float32)
    q_ref[...] = jnp.dot(x, wc_ref[...], preferred_element_type=jnp.float32) + bc_ref[...]


def _proj(x, wrow, wcol, bcol):
    return pl.pallas_call(
        _proj_body,
        out_shape=(
            jax.ShapeDtypeStruct((N, D), jnp.float32),
            jax.ShapeDtypeStruct((N, D), jnp.float32),
        ),
    )(x, wrow, wcol, bcol)


def _gather_body(p_hbm, q_hbm, row_hbm, col_hbm, gp_hbm, gq_hbm,
                 ridx_v, cidx_v, gp_v, gq_v, sem_r, sem_c):
    wid = lax.axis_index("s") * NC + lax.axis_index("c")
    per_w = E // NW
    base0 = wid * per_w

    def step(g, carry):
        base = base0 + g * CH
        pltpu.sync_copy(row_hbm.at[pl.ds(base, CH)], ridx_v)
        pltpu.sync_copy(col_hbm.at[pl.ds(base, CH)], cidx_v)
        cr = pltpu.async_copy(p_hbm.at[ridx_v], gp_v, sem_r)
        cc = pltpu.async_copy(q_hbm.at[cidx_v], gq_v, sem_c)
        cr.wait()
        cc.wait()
        pltpu.sync_copy(gp_v, gp_hbm.at[pl.ds(base, CH)])
        pltpu.sync_copy(gq_v, gq_hbm.at[pl.ds(base, CH)])
        return carry

    lax.fori_loop(0, per_w // CH, step, 0)


def _gather(p, q, row, col):
    mesh = plsc.VectorSubcoreMesh(core_axis_name="c", subcore_axis_name="s")
    return pl.kernel(
        _gather_body,
        out_type=(
            jax.ShapeDtypeStruct((E, D), jnp.float32),
            jax.ShapeDtypeStruct((E, D), jnp.float32),
        ),
        mesh=mesh,
        scratch_types=[
            pltpu.VMEM((CH,), jnp.int32),
            pltpu.VMEM((CH,), jnp.int32),
            pltpu.VMEM((CH, D), jnp.float32),
            pltpu.VMEM((CH, D), jnp.float32),
            pltpu.SemaphoreType.DMA,
            pltpu.SemaphoreType.DMA,
        ],
    )(p, q, row, col)


_EB = 6400


def _scores_body(gp_ref, gq_ref, w20_ref, w21_ref, s_ref):
    t = jnp.tanh(gp_ref[...] + gq_ref[...])
    s_ref[0, :] = jnp.sum(t[:, :H] * w20_ref[...], axis=1)
    s_ref[1, :] = jnp.sum(t[:, H:] * w21_ref[...], axis=1)


def _scores(gp, gq, w20, w21):
    return pl.pallas_call(
        _scores_body,
        grid=(E // _EB,),
        in_specs=[
            pl.BlockSpec((_EB, D), lambda i: (i, 0)),
            pl.BlockSpec((_EB, D), lambda i: (i, 0)),
            pl.BlockSpec((H,), lambda i: (0,)),
            pl.BlockSpec((H,), lambda i: (0,)),
        ],
        out_specs=pl.BlockSpec((2, _EB), lambda i: (0, i)),
        out_shape=jax.ShapeDtypeStruct((2, E), jnp.float32),
    )(gp, gq, w20, w21)


def _softmax_body(s_ref, w_ref):
    s = s_ref[...]
    m = jnp.max(s, axis=1, keepdims=True)
    e = jnp.exp(s - m)
    w_ref[...] = e / jnp.sum(e, axis=1, keepdims=True)


def _softmax(s):
    return pl.pallas_call(
        _softmax_body,
        out_shape=jax.ShapeDtypeStruct((2, E), jnp.float32),
    )(s)


NPAD = 10240


def _agg_body(x_hbm, row_hbm, col_hbm, w_hbm, out_hbm,
              ridx_v, cidx_v, w_v, rows_v, acc, sem):
    c = lax.axis_index("c")
    s = lax.axis_index("s")
    per_t = E // NS
    rows_per_t = NPAD // NS
    base0 = s * per_t

    def zero_buf(i, carry):
        for j in range(D // 16):
            rows_v[i, pl.ds(j * 16, 16)] = jnp.zeros((16,), jnp.float32)
        return carry

    lax.fori_loop(0, CH, zero_buf, 0)
    for k in range(rows_per_t // CH):
        pltpu.sync_copy(rows_v, acc.at[pl.ds(s * rows_per_t + k * CH, CH)])
    plsc.subcore_barrier()

    def step(g, carry):
        base = base0 + g * CH
        pltpu.sync_copy(col_hbm.at[pl.ds(base, CH)], cidx_v)
        pltpu.sync_copy(w_hbm.at[pl.ds(c * E + base, CH)], w_v)
        pltpu.async_copy(x_hbm.at[cidx_v], rows_v, sem).wait()

        def scale_group(g16, carry2):
            wvec = w_v[pl.ds(g16 * 16, 16)]
            for i in range(16):
                e = g16 * 16 + i
                wi = wvec[i]
                for j in range(D // 16):
                    sl = pl.ds(j * 16, 16)
                    rows_v[e, sl] = rows_v[e, sl] * wi
            return carry2

        lax.fori_loop(0, CH // 16, scale_group, 0)
        pltpu.sync_copy(row_hbm.at[pl.ds(base, CH)], ridx_v)
        pltpu.sync_copy(rows_v, acc.at[ridx_v], add=True)
        return carry

    lax.fori_loop(0, per_t // CH, step, 0)
    plsc.subcore_barrier()
    pltpu.sync_copy(acc.at[pl.ds(s * rows_per_t, rows_per_t)],
                    out_hbm.at[c, pl.ds(s * rows_per_t, rows_per_t)])


def _aggregate(x, row, col, w):
    mesh = plsc.VectorSubcoreMesh(core_axis_name="c", subcore_axis_name="s")
    return pl.kernel(
        _agg_body,
        out_type=jax.ShapeDtypeStruct((2, NPAD, D), jnp.float32),
        mesh=mesh,
        scratch_types=[
            pltpu.VMEM((CH,), jnp.int32),
            pltpu.VMEM((CH,), jnp.int32),
            pltpu.VMEM((CH,), jnp.float32),
            pltpu.VMEM((CH, D), jnp.float32),
            pltpu.VMEM_SHARED((NPAD, D), jnp.float32),
            pltpu.SemaphoreType.DMA,
        ],
    )(x, row, col, w)


def _tail_body(h0_ref, h1_ref, cw0_ref, cb0_ref, cw1_ref, cb1_ref,
               g0_ref, b0_ref, g1_ref, b1_ref, clw_ref, clb_ref, o_ref):
    h = jnp.concatenate([h0_ref[...], h1_ref[...]], axis=-1)

    def bn_relu(v, g, b):
        m = jnp.mean(v, axis=0, keepdims=True)
        var = jnp.mean((v - m) ** 2, axis=0, keepdims=True)
        return jnp.maximum((v - m) / jnp.sqrt(var + 1e-5) * g + b, 0.0)

    h = jnp.dot(h, cw0_ref[...], preferred_element_type=jnp.float32) + cb0_ref[...]
    h = bn_relu(h, g0_ref[...], b0_ref[...])
    h = jnp.dot(h, cw1_ref[...], preferred_element_type=jnp.float32) + cb1_ref[...]
    h = bn_relu(h, g1_ref[...], b1_ref[...])
    o_ref[...] = jnp.dot(h, clw_ref[...], preferred_element_type=jnp.float32) + clb_ref[...]


def _tail(h0, h1, cw0, cb0, cw1, cb1, g0, b0, g1, b1, clw, clb):
    return pl.pallas_call(
        _tail_body,
        out_shape=jax.ShapeDtypeStruct((N, 2), jnp.float32),
    )(h0, h1, cw0, cb0, cw1, cb1, g0, b0, g1, b1, clw, clb)


def kernel(x, aw1_0, ab1_0, aw2_0, ab2_0, aw1_1, ab1_1, aw2_1, ab2_1,
           cw0, cb0, cw1, cb1, bn_g0, bn_b0, bn_g1, bn_b1, clw, clb,
           edge_index):
    row = edge_index[0]
    col = edge_index[1]
    wrow = jnp.concatenate([aw1_0[:D], aw1_1[:D]], axis=1)
    wcol = jnp.concatenate([aw1_0[D:], aw1_1[D:]], axis=1)
    bcol = jnp.concatenate([ab1_0, ab1_1])

    p, q = _proj(x, wrow, wcol, bcol)
    gp, gq = _gather(p, q, row, col)
    s = _scores(gp, gq, aw2_0[:, 0], aw2_1[:, 0])
    w = _softmax(s)
    hagg = _aggregate(x, row, col, w.reshape(-1))[:, :N, :]
    return _tail(hagg[0], hagg[1], cw0, cb0, cw1, cb1,
                 bn_g0, bn_b0, bn_g1, bn_b1, clw, clb)


# speedup vs baseline: 2.8469x; 1.2790x over previous
import functools

import jax
import jax.numpy as jnp
from jax import lax
from jax.experimental import pallas as pl
from jax.experimental.pallas import tpu as pltpu, tpu_sc as plsc

N = 10000
E = 320000
D = 128
H = 64

_SC_INFO = plsc.get_sparse_core_info()
NC = _SC_INFO.num_cores
NS = _SC_INFO.num_subcores
NW = NC * NS
CH = 80


def _proj_body(x_ref, wr_ref, wc_ref, bc_ref, p_ref, q_ref):
    x = x_ref[...]
    p_ref[...] = jnp.dot(x, wr_ref[...], preferred_element_type=jnp.float32)
    q_ref[...] = jnp.dot(x, wc_ref[...], preferred_element_type=jnp.float32) + bc_ref[...]


def _proj(x, wrow, wcol, bcol):
    return pl.pallas_call(
        _proj_body,
        out_shape=(
            jax.ShapeDtypeStruct((N, D), jnp.float32),
            jax.ShapeDtypeStruct((N, D), jnp.float32),
        ),
    )(x, wrow, wcol, bcol)


CHG = 80


def _gather_body(p_hbm, q_hbm, row_hbm, col_hbm, t_hbm,
                 ridx0, cidx0, ridx1, cidx1, gp0, gq0, gp1, gq1,
                 semr0, semc0, semr1, semc1, semw0, semw1):
    wid = lax.axis_index("s") * NC + lax.axis_index("c")
    per_w = E // NW
    base0 = wid * per_w

    def add_rows(gp, gq):
        def row(e, carry):
            for j in range(D // 16):
                sl = pl.ds(j * 16, 16)
                gp[e, sl] = gp[e, sl] + gq[e, sl]
            return carry

        lax.fori_loop(0, CHG, row, 0)

    def pair(i, carry):
        b0 = base0 + (2 * i) * CHG
        b1 = b0 + CHG
        pltpu.sync_copy(row_hbm.at[pl.ds(b0, CHG)], ridx0)
        pltpu.sync_copy(col_hbm.at[pl.ds(b0, CHG)], cidx0)
        r0 = pltpu.async_copy(p_hbm.at[ridx0], gp0, semr0)
        c0 = pltpu.async_copy(q_hbm.at[cidx0], gq0, semc0)
        pltpu.sync_copy(row_hbm.at[pl.ds(b1, CHG)], ridx1)
        pltpu.sync_copy(col_hbm.at[pl.ds(b1, CHG)], cidx1)
        r1 = pltpu.async_copy(p_hbm.at[ridx1], gp1, semr1)
        c1 = pltpu.async_copy(q_hbm.at[cidx1], gq1, semc1)
        r0.wait()
        c0.wait()
        add_rows(gp0, gq0)
        w0 = pltpu.async_copy(gp0, t_hbm.at[pl.ds(b0, CHG)], semw0)
        r1.wait()
        c1.wait()
        add_rows(gp1, gq1)
        w1 = pltpu.async_copy(gp1, t_hbm.at[pl.ds(b1, CHG)], semw1)
        w0.wait()
        w1.wait()
        return carry

    n_chunks = per_w // CHG
    lax.fori_loop(0, n_chunks // 2, pair, 0)

    bt = base0 + (n_chunks - 1) * CHG
    pltpu.sync_copy(row_hbm.at[pl.ds(bt, CHG)], ridx0)
    pltpu.sync_copy(col_hbm.at[pl.ds(bt, CHG)], cidx0)
    rt = pltpu.async_copy(p_hbm.at[ridx0], gp0, semr0)
    ct = pltpu.async_copy(q_hbm.at[cidx0], gq0, semc0)
    rt.wait()
    ct.wait()
    add_rows(gp0, gq0)
    pltpu.sync_copy(gp0, t_hbm.at[pl.ds(bt, CHG)])


def _gather(p, q, row, col):
    mesh = plsc.VectorSubcoreMesh(core_axis_name="c", subcore_axis_name="s")
    return pl.kernel(
        _gather_body,
        out_type=jax.ShapeDtypeStruct((E, D), jnp.float32),
        mesh=mesh,
        scratch_types=[
            pltpu.VMEM((CHG,), jnp.int32),
            pltpu.VMEM((CHG,), jnp.int32),
            pltpu.VMEM((CHG,), jnp.int32),
            pltpu.VMEM((CHG,), jnp.int32),
            pltpu.VMEM((CHG, D), jnp.float32),
            pltpu.VMEM((CHG, D), jnp.float32),
            pltpu.VMEM((CHG, D), jnp.float32),
            pltpu.VMEM((CHG, D), jnp.float32),
            pltpu.SemaphoreType.DMA,
            pltpu.SemaphoreType.DMA,
            pltpu.SemaphoreType.DMA,
            pltpu.SemaphoreType.DMA,
            pltpu.SemaphoreType.DMA,
            pltpu.SemaphoreType.DMA,
        ],
    )(p, q, row, col)


_EB = 6400


def _scores_body(t_ref, w20_ref, w21_ref, s_ref):
    t = jnp.tanh(t_ref[...])
    s_ref[0, :] = jnp.sum(t[:, :H] * w20_ref[...], axis=1)
    s_ref[1, :] = jnp.sum(t[:, H:] * w21_ref[...], axis=1)


def _scores(t, w20, w21):
    return pl.pallas_call(
        _scores_body,
        grid=(E // _EB,),
        in_specs=[
            pl.BlockSpec((_EB, D), lambda i: (i, 0)),
            pl.BlockSpec((H,), lambda i: (0,)),
            pl.BlockSpec((H,), lambda i: (0,)),
        ],
        out_specs=pl.BlockSpec((2, _EB), lambda i: (0, i)),
        out_shape=jax.ShapeDtypeStruct((2, E), jnp.float32),
    )(t, w20, w21)


def _softmax_body(s_ref, w_ref):
    s = s_ref[...]
    m = jnp.max(s, axis=1, keepdims=True)
    e = jnp.exp(s - m)
    w_ref[...] = e / jnp.sum(e, axis=1, keepdims=True)


def _softmax(s):
    return pl.pallas_call(
        _softmax_body,
        out_shape=jax.ShapeDtypeStruct((2, E), jnp.float32),
    )(s)


NPAD = 10240


def _agg_body(x_hbm, row_hbm, col_hbm, w_hbm, out_hbm,
              ridx0, cidx0, w0, ridx1, cidx1, w1, x0, x1, acc, sem0, sem1):
    c = lax.axis_index("c")
    s = lax.axis_index("s")
    per_t = E // NS
    rows_per_t = NPAD // NS
    base0 = s * per_t

    def zero_buf(i, carry):
        for j in range(D // 16):
            x0[i, pl.ds(j * 16, 16)] = jnp.zeros((16,), jnp.float32)
        return carry

    lax.fori_loop(0, CH, zero_buf, 0)
    for k in range(rows_per_t // CH):
        pltpu.sync_copy(x0, acc.at[pl.ds(s * rows_per_t + k * CH, CH)])
    plsc.subcore_barrier()

    def scale_scatter(xb, wb, rb, base):
        def scale_group(g16, carry2):
            wvec = wb[pl.ds(g16 * 16, 16)]
            for i in range(16):
                e = g16 * 16 + i
                wi = wvec[i]
                for j in range(D // 16):
                    sl = pl.ds(j * 16, 16)
                    xb[e, sl] = xb[e, sl] * wi
            return carry2

        lax.fori_loop(0, CH // 16, scale_group, 0)
        pltpu.sync_copy(row_hbm.at[pl.ds(base, CH)], rb)
        pltpu.sync_copy(xb, acc.at[rb], add=True)

    def pair(g, carry):
        b0 = base0 + (2 * g) * CH
        b1 = b0 + CH
        pltpu.sync_copy(col_hbm.at[pl.ds(b0, CH)], cidx0)
        pltpu.sync_copy(w_hbm.at[pl.ds(c * E + b0, CH)], w0)
        g0 = pltpu.async_copy(x_hbm.at[cidx0], x0, sem0)
        pltpu.sync_copy(col_hbm.at[pl.ds(b1, CH)], cidx1)
        pltpu.sync_copy(w_hbm.at[pl.ds(c * E + b1, CH)], w1)
        g1 = pltpu.async_copy(x_hbm.at[cidx1], x1, sem1)
        g0.wait()
        scale_scatter(x0, w0, ridx0, b0)
        g1.wait()
        scale_scatter(x1, w1, ridx1, b1)
        return carry

    lax.fori_loop(0, per_t // (2 * CH), pair, 0)
    plsc.subcore_barrier()
    pltpu.sync_copy(acc.at[pl.ds(s * rows_per_t, rows_per_t)],
                    out_hbm.at[c, pl.ds(s * rows_per_t, rows_per_t)])


def _aggregate(x, row, col, w):
    mesh = plsc.VectorSubcoreMesh(core_axis_name="c", subcore_axis_name="s")
    return pl.kernel(
        _agg_body,
        out_type=jax.ShapeDtypeStruct((2, NPAD, D), jnp.float32),
        mesh=mesh,
        scratch_types=[
            pltpu.VMEM((CH,), jnp.int32),
            pltpu.VMEM((CH,), jnp.int32),
            pltpu.VMEM((CH,), jnp.float32),
            pltpu.VMEM((CH,), jnp.int32),
            pltpu.VMEM((CH,), jnp.int32),
            pltpu.VMEM((CH,), jnp.float32),
            pltpu.VMEM((CH, D), jnp.float32),
            pltpu.VMEM((CH, D), jnp.float32),
            pltpu.VMEM_SHARED((NPAD, D), jnp.float32),
            pltpu.SemaphoreType.DMA,
            pltpu.SemaphoreType.DMA,
        ],
    )(x, row, col, w)


def _tail_body(h0_ref, h1_ref, cw0_ref, cb0_ref, cw1_ref, cb1_ref,
               g0_ref, b0_ref, g1_ref, b1_ref, clw_ref, clb_ref, o_ref):
    h = jnp.concatenate([h0_ref[...], h1_ref[...]], axis=-1)

    def bn_relu(v, g, b):
        m = jnp.mean(v, axis=0, keepdims=True)
        var = jnp.mean((v - m) ** 2, axis=0, keepdims=True)
        return jnp.maximum((v - m) / jnp.sqrt(var + 1e-5) * g + b, 0.0)

    h = jnp.dot(h, cw0_ref[...], preferred_element_type=jnp.float32) + cb0_ref[...]
    h = bn_relu(h, g0_ref[...], b0_ref[...])
    h = jnp.dot(h, cw1_ref[...], preferred_element_type=jnp.float32) + cb1_ref[...]
    h = bn_relu(h, g1_ref[...], b1_ref[...])
    o_ref[...] = jnp.dot(h, clw_ref[...], preferred_element_type=jnp.float32) + clb_ref[...]


def _tail(h0, h1, cw0, cb0, cw1, cb1, g0, b0, g1, b1, clw, clb):
    return pl.pallas_call(
        _tail_body,
        out_shape=jax.ShapeDtypeStruct((N, 2), jnp.float32),
    )(h0, h1, cw0, cb0, cw1, cb1, g0, b0, g1, b1, clw, clb)


def kernel(x, aw1_0, ab1_0, aw2_0, ab2_0, aw1_1, ab1_1, aw2_1, ab2_1,
           cw0, cb0, cw1, cb1, bn_g0, bn_b0, bn_g1, bn_b1, clw, clb,
           edge_index):
    row = edge_index[0]
    col = edge_index[1]
    wrow = jnp.concatenate([aw1_0[:D], aw1_1[:D]], axis=1)
    wcol = jnp.concatenate([aw1_0[D:], aw1_1[D:]], axis=1)
    bcol = jnp.concatenate([ab1_0, ab1_1])

    p, q = _proj(x, wrow, wcol, bcol)
    t = _gather(p, q, row, col)
    s = _scores(t, aw2_0[:, 0], aw2_1[:, 0])
    w = _softmax(s)
    hagg = _aggregate(x, row, col, w.reshape(-1))[:, :N, :]
    return _tail(hagg[0], hagg[1], cw0, cb0, cw1, cb1,
                 bn_g0, bn_b0, bn_g1, bn_b1, clw, clb)


# speedup vs baseline: 3.4892x; 1.2256x over previous
import functools

import jax
import jax.numpy as jnp
from jax import lax
from jax.experimental import pallas as pl
from jax.experimental.pallas import tpu as pltpu, tpu_sc as plsc

N = 10000
E = 320000
D = 128
H = 64

_SC_INFO = plsc.get_sparse_core_info()
NC = _SC_INFO.num_cores
NS = _SC_INFO.num_subcores
NW = NC * NS
CH = 80


def _proj_body(x_ref, wr_ref, wc_ref, bc_ref, p_ref, q_ref):
    x = x_ref[...]
    p_ref[...] = jnp.dot(x, wr_ref[...], preferred_element_type=jnp.float32)
    q_ref[...] = jnp.dot(x, wc_ref[...], preferred_element_type=jnp.float32) + bc_ref[...]


def _proj(x, wrow, wcol, bcol):
    return pl.pallas_call(
        _proj_body,
        out_shape=(
            jax.ShapeDtypeStruct((N, D), jnp.float32),
            jax.ShapeDtypeStruct((N, D), jnp.float32),
        ),
    )(x, wrow, wcol, bcol)


CHG = 80


def _gather_body(p_hbm, q_hbm, row2_hbm, col2_hbm, t_hbm,
                 rows_all, cols_all, gp0, gq0, gp1, gq1,
                 semr0, semc0, semr1, semc1, semw0, semw1):
    wid = lax.axis_index("s") * NC + lax.axis_index("c")
    per_w = E // NW
    n_chunks = per_w // CHG
    base0 = wid * per_w

    pltpu.sync_copy(row2_hbm.at[wid], rows_all)
    pltpu.sync_copy(col2_hbm.at[wid], cols_all)

    def add_rows(gp, gq):
        def row(e, carry):
            for j in range(D // 16):
                sl = pl.ds(j * 16, 16)
                gp[e, sl] = gp[e, sl] + gq[e, sl]
            return carry

        lax.fori_loop(0, CHG, row, 0)

    def pair(i, carry):
        b0 = base0 + (2 * i) * CHG
        b1 = b0 + CHG
        r0 = pltpu.async_copy(p_hbm.at[rows_all.at[2 * i]], gp0, semr0)
        c0 = pltpu.async_copy(q_hbm.at[cols_all.at[2 * i]], gq0, semc0)
        r1 = pltpu.async_copy(p_hbm.at[rows_all.at[2 * i + 1]], gp1, semr1)
        c1 = pltpu.async_copy(q_hbm.at[cols_all.at[2 * i + 1]], gq1, semc1)
        r0.wait()
        c0.wait()
        add_rows(gp0, gq0)
        w0 = pltpu.async_copy(gp0, t_hbm.at[pl.ds(b0, CHG)], semw0)
        r1.wait()
        c1.wait()
        add_rows(gp1, gq1)
        w1 = pltpu.async_copy(gp1, t_hbm.at[pl.ds(b1, CHG)], semw1)
        w0.wait()
        w1.wait()
        return carry

    lax.fori_loop(0, n_chunks // 2, pair, 0)

    bt = base0 + (n_chunks - 1) * CHG
    rt = pltpu.async_copy(p_hbm.at[rows_all.at[n_chunks - 1]], gp0, semr0)
    ct = pltpu.async_copy(q_hbm.at[cols_all.at[n_chunks - 1]], gq0, semc0)
    rt.wait()
    ct.wait()
    add_rows(gp0, gq0)
    pltpu.sync_copy(gp0, t_hbm.at[pl.ds(bt, CHG)])


def _gather(p, q, row2, col2):
    mesh = plsc.VectorSubcoreMesh(core_axis_name="c", subcore_axis_name="s")
    nck = E // NW // CHG
    return pl.kernel(
        _gather_body,
        out_type=jax.ShapeDtypeStruct((E, D), jnp.float32),
        mesh=mesh,
        scratch_types=[
            pltpu.VMEM((nck, CHG), jnp.int32),
            pltpu.VMEM((nck, CHG), jnp.int32),
            pltpu.VMEM((CHG, D), jnp.float32),
            pltpu.VMEM((CHG, D), jnp.float32),
            pltpu.VMEM((CHG, D), jnp.float32),
            pltpu.VMEM((CHG, D), jnp.float32),
            pltpu.SemaphoreType.DMA,
            pltpu.SemaphoreType.DMA,
            pltpu.SemaphoreType.DMA,
            pltpu.SemaphoreType.DMA,
            pltpu.SemaphoreType.DMA,
            pltpu.SemaphoreType.DMA,
        ],
    )(p, q, row2, col2)


_EB = 6400


def _scores_body(t_ref, w20_ref, w21_ref, s_ref):
    t = jnp.tanh(t_ref[...])
    s_ref[0, :] = jnp.sum(t[:, :H] * w20_ref[...], axis=1)
    s_ref[1, :] = jnp.sum(t[:, H:] * w21_ref[...], axis=1)


def _scores(t, w20, w21):
    return pl.pallas_call(
        _scores_body,
        grid=(E // _EB,),
        in_specs=[
            pl.BlockSpec((_EB, D), lambda i: (i, 0)),
            pl.BlockSpec((H,), lambda i: (0,)),
            pl.BlockSpec((H,), lambda i: (0,)),
        ],
        out_specs=pl.BlockSpec((2, _EB), lambda i: (0, i)),
        out_shape=jax.ShapeDtypeStruct((2, E), jnp.float32),
    )(t, w20, w21)


def _softmax_body(s_ref, w_ref):
    s = s_ref[...]
    m = jnp.max(s, axis=1, keepdims=True)
    e = jnp.exp(s - m)
    w_ref[...] = e / jnp.sum(e, axis=1, keepdims=True)


def _softmax(s):
    return pl.pallas_call(
        _softmax_body,
        out_shape=jax.ShapeDtypeStruct((2, E), jnp.float32),
    )(s)


NPAD = 10240


SB = 50


def _agg_body(x_hbm, row3_hbm, col3_hbm, w3_hbm, out_hbm,
              rows_all, cols_all, w_all, x0, x1, acc,
              sem0, sem1, semsc0, semsc1):
    c = lax.axis_index("c")
    s = lax.axis_index("s")
    per_t = E // NS
    n_chunks = per_t // CH
    nsb = n_chunks // SB
    rows_per_t = NPAD // NS

    def zero_buf(i, carry):
        for j in range(D // 16):
            x0[i, pl.ds(j * 16, 16)] = jnp.zeros((16,), jnp.float32)
        return carry

    lax.fori_loop(0, CH, zero_buf, 0)
    for k in range(rows_per_t // CH):
        pltpu.sync_copy(x0, acc.at[pl.ds(s * rows_per_t + k * CH, CH)])
    plsc.subcore_barrier()

    def scale(xb, g):
        def scale_group(g16, carry2):
            wvec = w_all[g, pl.ds(g16 * 16, 16)]
            for i in range(16):
                e = g16 * 16 + i
                wi = wvec[i]
                for j in range(D // 16):
                    sl = pl.ds(j * 16, 16)
                    xb[e, sl] = xb[e, sl] * wi
            return carry2

        lax.fori_loop(0, CH // 16, scale_group, 0)

    def pair(i, carry):
        g0i = 2 * i
        g1i = 2 * i + 1
        g0 = pltpu.async_copy(x_hbm.at[cols_all.at[g0i]], x0, sem0)
        g1 = pltpu.async_copy(x_hbm.at[cols_all.at[g1i]], x1, sem1)
        g0.wait()
        scale(x0, g0i)
        sc0 = pltpu.async_copy(x0, acc.at[rows_all.at[g0i]], semsc0, add=True)
        g1.wait()
        scale(x1, g1i)
        sc1 = pltpu.async_copy(x1, acc.at[rows_all.at[g1i]], semsc1, add=True)
        sc0.wait()
        sc1.wait()
        return carry

    def sb_loop(sb, carry):
        pltpu.sync_copy(row3_hbm.at[s * nsb + sb], rows_all)
        pltpu.sync_copy(col3_hbm.at[s * nsb + sb], cols_all)
        pltpu.sync_copy(w3_hbm.at[(c * NS + s) * nsb + sb], w_all)
        lax.fori_loop(0, SB // 2, pair, 0)
        return carry

    lax.fori_loop(0, nsb, sb_loop, 0)
    plsc.subcore_barrier()
    pltpu.sync_copy(acc.at[pl.ds(s * rows_per_t, rows_per_t)],
                    out_hbm.at[c, pl.ds(s * rows_per_t, rows_per_t)])


def _aggregate(x, row3, col3, w3):
    mesh = plsc.VectorSubcoreMesh(core_axis_name="c", subcore_axis_name="s")
    return pl.kernel(
        _agg_body,
        out_type=jax.ShapeDtypeStruct((2, NPAD, D), jnp.float32),
        mesh=mesh,
        scratch_types=[
            pltpu.VMEM((SB, CH), jnp.int32),
            pltpu.VMEM((SB, CH), jnp.int32),
            pltpu.VMEM((SB, CH), jnp.float32),
            pltpu.VMEM((CH, D), jnp.float32),
            pltpu.VMEM((CH, D), jnp.float32),
            pltpu.VMEM_SHARED((NPAD, D), jnp.float32),
            pltpu.SemaphoreType.DMA,
            pltpu.SemaphoreType.DMA,
            pltpu.SemaphoreType.DMA,
            pltpu.SemaphoreType.DMA,
        ],
    )(x, row3, col3, w3)


def _tail_body(h0_ref, h1_ref, cw0_ref, cb0_ref, cw1_ref, cb1_ref,
               g0_ref, b0_ref, g1_ref, b1_ref, clw_ref, clb_ref, o_ref):
    h = jnp.concatenate([h0_ref[...], h1_ref[...]], axis=-1)

    def bn_relu(v, g, b):
        m = jnp.mean(v, axis=0, keepdims=True)
        var = jnp.mean((v - m) ** 2, axis=0, keepdims=True)
        return jnp.maximum((v - m) / jnp.sqrt(var + 1e-5) * g + b, 0.0)

    h = jnp.dot(h, cw0_ref[...], preferred_element_type=jnp.float32) + cb0_ref[...]
    h = bn_relu(h, g0_ref[...], b0_ref[...])
    h = jnp.dot(h, cw1_ref[...], preferred_element_type=jnp.float32) + cb1_ref[...]
    h = bn_relu(h, g1_ref[...], b1_ref[...])
    o_ref[...] = jnp.dot(h, clw_ref[...], preferred_element_type=jnp.float32) + clb_ref[...]


def _tail(h0, h1, cw0, cb0, cw1, cb1, g0, b0, g1, b1, clw, clb):
    return pl.pallas_call(
        _tail_body,
        out_shape=jax.ShapeDtypeStruct((N, 2), jnp.float32),
    )(h0, h1, cw0, cb0, cw1, cb1, g0, b0, g1, b1, clw, clb)


def kernel(x, aw1_0, ab1_0, aw2_0, ab2_0, aw1_1, ab1_1, aw2_1, ab2_1,
           cw0, cb0, cw1, cb1, bn_g0, bn_b0, bn_g1, bn_b1, clw, clb,
           edge_index):
    rowg = edge_index[0].reshape(NW, E // NW // CHG, CHG)
    colg = edge_index[1].reshape(NW, E // NW // CHG, CHG)
    nsb = E // NS // CH // SB
    rowa = edge_index[0].reshape(NS * nsb, SB, CH)
    cola = edge_index[1].reshape(NS * nsb, SB, CH)
    wrow = jnp.concatenate([aw1_0[:D], aw1_1[:D]], axis=1)
    wcol = jnp.concatenate([aw1_0[D:], aw1_1[D:]], axis=1)
    bcol = jnp.concatenate([ab1_0, ab1_1])

    p, q = _proj(x, wrow, wcol, bcol)
    t = _gather(p, q, rowg, colg)
    s = _scores(t, aw2_0[:, 0], aw2_1[:, 0])
    w = _softmax(s)
    hagg = _aggregate(x, rowa, cola,
                      w.reshape(2 * NS * nsb, SB, CH))[:, :N, :]
    return _tail(hagg[0], hagg[1], cw0, cb0, cw1, cb1,
                 bn_g0, bn_b0, bn_g1, bn_b1, clw, clb)


# speedup vs baseline: 3.5216x; 1.0093x over previous
import functools

import jax
import jax.numpy as jnp
from jax import lax
from jax.experimental import pallas as pl
from jax.experimental.pallas import tpu as pltpu, tpu_sc as plsc

N = 10000
E = 320000
D = 128
H = 64

_SC_INFO = plsc.get_sparse_core_info()
NC = _SC_INFO.num_cores
NS = _SC_INFO.num_subcores
NW = NC * NS
CH = 80


def _proj_body(x_ref, wr_ref, wc_ref, bc_ref, p_ref, q_ref):
    x = x_ref[...]
    p_ref[...] = jnp.dot(x, wr_ref[...], preferred_element_type=jnp.float32)
    q_ref[...] = jnp.dot(x, wc_ref[...], preferred_element_type=jnp.float32) + bc_ref[...]


def _proj(x, wrow, wcol, bcol):
    return pl.pallas_call(
        _proj_body,
        out_shape=(
            jax.ShapeDtypeStruct((N, D), jnp.float32),
            jax.ShapeDtypeStruct((N, D), jnp.float32),
        ),
    )(x, wrow, wcol, bcol)


CHG = 80


def _gather_body(p_hbm, q_hbm, row2_hbm, col2_hbm, t_hbm,
                 rows_all, cols_all, gp0, gq0, gp1, gq1,
                 semr0, semc0, semr1, semc1, semw0, semw1):
    wid = lax.axis_index("s") * NC + lax.axis_index("c")
    per_w = E // NW
    n_chunks = per_w // CHG
    base0 = wid * per_w

    pltpu.sync_copy(row2_hbm.at[wid], rows_all)
    pltpu.sync_copy(col2_hbm.at[wid], cols_all)

    def add_rows(gp, gq):
        def row(e, carry):
            for j in range(D // 16):
                sl = pl.ds(j * 16, 16)
                gp[e, sl] = gp[e, sl] + gq[e, sl]
            return carry

        lax.fori_loop(0, CHG, row, 0)

    def pair(i, carry):
        b0 = base0 + (2 * i) * CHG
        b1 = b0 + CHG
        r0 = pltpu.async_copy(p_hbm.at[rows_all.at[2 * i]], gp0, semr0)
        c0 = pltpu.async_copy(q_hbm.at[cols_all.at[2 * i]], gq0, semc0)
        r1 = pltpu.async_copy(p_hbm.at[rows_all.at[2 * i + 1]], gp1, semr1)
        c1 = pltpu.async_copy(q_hbm.at[cols_all.at[2 * i + 1]], gq1, semc1)
        r0.wait()
        c0.wait()
        add_rows(gp0, gq0)
        w0 = pltpu.async_copy(gp0, t_hbm.at[pl.ds(b0, CHG)], semw0)
        r1.wait()
        c1.wait()
        add_rows(gp1, gq1)
        w1 = pltpu.async_copy(gp1, t_hbm.at[pl.ds(b1, CHG)], semw1)
        w0.wait()
        w1.wait()
        return carry

    lax.fori_loop(0, n_chunks // 2, pair, 0)

    bt = base0 + (n_chunks - 1) * CHG
    rt = pltpu.async_copy(p_hbm.at[rows_all.at[n_chunks - 1]], gp0, semr0)
    ct = pltpu.async_copy(q_hbm.at[cols_all.at[n_chunks - 1]], gq0, semc0)
    rt.wait()
    ct.wait()
    add_rows(gp0, gq0)
    pltpu.sync_copy(gp0, t_hbm.at[pl.ds(bt, CHG)])


def _gather(p, q, row2, col2):
    mesh = plsc.VectorSubcoreMesh(core_axis_name="c", subcore_axis_name="s")
    nck = E // NW // CHG
    return pl.kernel(
        _gather_body,
        out_type=jax.ShapeDtypeStruct((E, D), jnp.float32),
        mesh=mesh,
        scratch_types=[
            pltpu.VMEM((nck, CHG), jnp.int32),
            pltpu.VMEM((nck, CHG), jnp.int32),
            pltpu.VMEM((CHG, D), jnp.float32),
            pltpu.VMEM((CHG, D), jnp.float32),
            pltpu.VMEM((CHG, D), jnp.float32),
            pltpu.VMEM((CHG, D), jnp.float32),
            pltpu.SemaphoreType.DMA,
            pltpu.SemaphoreType.DMA,
            pltpu.SemaphoreType.DMA,
            pltpu.SemaphoreType.DMA,
            pltpu.SemaphoreType.DMA,
            pltpu.SemaphoreType.DMA,
        ],
    )(p, q, row2, col2)


_EB = 6400


def _scores_body(t_ref, w20_ref, w21_ref, s_ref):
    t = jnp.tanh(t_ref[...])
    s_ref[0, :] = jnp.sum(t[:, :H] * w20_ref[...], axis=1)
    s_ref[1, :] = jnp.sum(t[:, H:] * w21_ref[...], axis=1)


def _scores(t, w20, w21):
    return pl.pallas_call(
        _scores_body,
        grid=(E // _EB,),
        in_specs=[
            pl.BlockSpec((_EB, D), lambda i: (i, 0)),
            pl.BlockSpec((H,), lambda i: (0,)),
            pl.BlockSpec((H,), lambda i: (0,)),
        ],
        out_specs=pl.BlockSpec((2, _EB), lambda i: (0, i)),
        out_shape=jax.ShapeDtypeStruct((2, E), jnp.float32),
    )(t, w20, w21)


def _softmax_body(s_ref, w_ref):
    s = s_ref[...]
    m = jnp.max(s, axis=1, keepdims=True)
    e = jnp.exp(s - m)
    w_ref[...] = e / jnp.sum(e, axis=1, keepdims=True)


def _softmax(s):
    return pl.pallas_call(
        _softmax_body,
        out_shape=jax.ShapeDtypeStruct((2, E), jnp.float32),
    )(s)


NPAD = 10240


SB = 50


def _agg_body(x_hbm, row3_hbm, col3_hbm, w3_hbm, out_hbm,
              rows_all, cols_all, w_all, x0, x1, acc,
              sem0, sem1, semsc0, semsc1):
    c = lax.axis_index("c")
    s = lax.axis_index("s")
    per_t = E // NS
    n_chunks = per_t // CH
    nsb = n_chunks // SB
    rows_per_t = NPAD // NS

    def zero_buf(i, carry):
        for j in range(D // 16):
            x0[i, pl.ds(j * 16, 16)] = jnp.zeros((16,), jnp.float32)
        return carry

    lax.fori_loop(0, CH, zero_buf, 0)
    for k in range(rows_per_t // CH):
        pltpu.sync_copy(x0, acc.at[pl.ds(s * rows_per_t + k * CH, CH)])
    plsc.subcore_barrier()

    def scale(xb, g):
        def scale_group(g16, carry2):
            wvec = w_all[g, pl.ds(g16 * 16, 16)]
            for i in range(16):
                e = g16 * 16 + i
                wi = wvec[i]
                for j in range(D // 16):
                    sl = pl.ds(j * 16, 16)
                    xb[e, sl] = xb[e, sl] * wi
            return carry2

        lax.fori_loop(0, CH // 16, scale_group, 0)

    def pair(i, carry):
        g0i = 2 * i
        g1i = 2 * i + 1
        g0 = pltpu.async_copy(x_hbm.at[cols_all.at[g0i]], x0, sem0)
        g1 = pltpu.async_copy(x_hbm.at[cols_all.at[g1i]], x1, sem1)
        g0.wait()
        scale(x0, g0i)
        sc0 = pltpu.async_copy(x0, acc.at[rows_all.at[g0i]], semsc0, add=True)
        g1.wait()
        scale(x1, g1i)
        sc1 = pltpu.async_copy(x1, acc.at[rows_all.at[g1i]], semsc1, add=True)
        sc0.wait()
        sc1.wait()
        return carry

    def sb_loop(sb, carry):
        pltpu.sync_copy(row3_hbm.at[s * nsb + sb], rows_all)
        pltpu.sync_copy(col3_hbm.at[s * nsb + sb], cols_all)
        pltpu.sync_copy(w3_hbm.at[(c * NS + s) * nsb + sb], w_all)
        lax.fori_loop(0, SB // 2, pair, 0)
        return carry

    lax.fori_loop(0, nsb, sb_loop, 0)
    plsc.subcore_barrier()
    pltpu.sync_copy(acc.at[pl.ds(s * rows_per_t, rows_per_t)],
                    out_hbm.at[c, pl.ds(s * rows_per_t, rows_per_t)])


def _aggregate(x, row3, col3, w3):
    mesh = plsc.VectorSubcoreMesh(core_axis_name="c", subcore_axis_name="s")
    return pl.kernel(
        _agg_body,
        out_type=jax.ShapeDtypeStruct((2, NPAD, D), jnp.float32),
        mesh=mesh,
        scratch_types=[
            pltpu.VMEM((SB, CH), jnp.int32),
            pltpu.VMEM((SB, CH), jnp.int32),
            pltpu.VMEM((SB, CH), jnp.float32),
            pltpu.VMEM((CH, D), jnp.float32),
            pltpu.VMEM((CH, D), jnp.float32),
            pltpu.VMEM_SHARED((NPAD, D), jnp.float32),
            pltpu.SemaphoreType.DMA,
            pltpu.SemaphoreType.DMA,
            pltpu.SemaphoreType.DMA,
            pltpu.SemaphoreType.DMA,
        ],
    )(x, row3, col3, w3)


def _tail_body(hagg_ref, cw0_ref, cb0_ref, cw1_ref, cb1_ref,
               g0_ref, b0_ref, g1_ref, b1_ref, clw_ref, clb_ref, o_ref):
    h = jnp.concatenate([hagg_ref[0, :N, :], hagg_ref[1, :N, :]], axis=-1)

    def bn_relu(v, g, b):
        m = jnp.mean(v, axis=0, keepdims=True)
        var = jnp.mean((v - m) ** 2, axis=0, keepdims=True)
        return jnp.maximum((v - m) / jnp.sqrt(var + 1e-5) * g + b, 0.0)

    h = jnp.dot(h, cw0_ref[...], preferred_element_type=jnp.float32) + cb0_ref[...]
    h = bn_relu(h, g0_ref[...], b0_ref[...])
    h = jnp.dot(h, cw1_ref[...], preferred_element_type=jnp.float32) + cb1_ref[...]
    h = bn_relu(h, g1_ref[...], b1_ref[...])
    o_ref[...] = jnp.dot(h, clw_ref[...], preferred_element_type=jnp.float32) + clb_ref[...]


def _tail(hagg, cw0, cb0, cw1, cb1, g0, b0, g1, b1, clw, clb):
    return pl.pallas_call(
        _tail_body,
        out_shape=jax.ShapeDtypeStruct((N, 2), jnp.float32),
    )(hagg, cw0, cb0, cw1, cb1, g0, b0, g1, b1, clw, clb)


def kernel(x, aw1_0, ab1_0, aw2_0, ab2_0, aw1_1, ab1_1, aw2_1, ab2_1,
           cw0, cb0, cw1, cb1, bn_g0, bn_b0, bn_g1, bn_b1, clw, clb,
           edge_index):
    rowg = edge_index[0].reshape(NW, E // NW // CHG, CHG)
    colg = edge_index[1].reshape(NW, E // NW // CHG, CHG)
    nsb = E // NS // CH // SB
    rowa = edge_index[0].reshape(NS * nsb, SB, CH)
    cola = edge_index[1].reshape(NS * nsb, SB, CH)
    wrow = jnp.concatenate([aw1_0[:D], aw1_1[:D]], axis=1)
    wcol = jnp.concatenate([aw1_0[D:], aw1_1[D:]], axis=1)
    bcol = jnp.concatenate([ab1_0, ab1_1])

    p, q = _proj(x, wrow, wcol, bcol)
    t = _gather(p, q, rowg, colg)
    s = _scores(t, aw2_0[:, 0], aw2_1[:, 0])
    w = _softmax(s)
    hagg = _aggregate(x, rowa, cola, w.reshape(2 * NS * nsb, SB, CH))
    return _tail(hagg, cw0, cb0, cw1, cb1,
                 bn_g0, bn_b0, bn_g1, bn_b1, clw, clb)
